# Initial kernel scaffold; baseline (speedup 1.0000x reference)
import jax
import jax.numpy as jnp
from jax.experimental import pallas as pl


def kernel(x, edge_index, batch, W0, b0, g0, be0, W1, b1, g1, be1, W2, b2, g2, be2, W_ih, W_hh, b_ih, b_hh, W_fc, b_fc):
    raise NotImplementedError("write your pallas kernel here")


# speedup vs baseline: 7.3432x; 7.3432x over previous
import functools

import jax
import jax.numpy as jnp
from jax import lax
from jax.experimental import pallas as pl
from jax.experimental.pallas import tpu as pltpu
from jax.experimental.pallas import tpu_sc as plsc

NC, NS = 2, 16
NW = NC * NS
CHUNK = 128
AGGR = 10112


def _sc_mesh():
    return plsc.VectorSubcoreMesh(core_axis_name="c", subcore_axis_name="s")


def _make_degree_kernel(cpt):
    @functools.partial(
        pl.kernel,
        out_type=jax.ShapeDtypeStruct((NW, AGGR), jnp.float32),
        mesh=_sc_mesh(),
        scratch_types=[
            pltpu.VMEM((cpt, CHUNK), jnp.int32),
            pltpu.VMEM((AGGR,), jnp.float32),
        ],
        compiler_params=pltpu.CompilerParams(needs_layout_passes=False),
    )
    def deg_kernel(dst_hbm, out_hbm, dstv, cnt):
        c = lax.axis_index("c")
        s = lax.axis_index("s")
        wid = c * NS + s
        pltpu.sync_copy(dst_hbm.at[pl.ds(wid * cpt, cpt)], dstv)

        def zero_body(i, carry):
            cnt[pl.ds(i * 16, 16)] = jnp.zeros((16,), jnp.float32)
            return carry

        lax.fori_loop(0, AGGR // 16, zero_body, 0)

        ones = jnp.full((16,), 1.0, jnp.float32)

        def count_body(r, carry):
            for j in range(CHUNK // 16):
                idx = dstv[r, pl.ds(j * 16, 16)]
                plsc.addupdate_scatter(cnt, [idx], ones)
            return carry

        lax.fori_loop(0, cpt, count_body, 0)
        pltpu.sync_copy(cnt, out_hbm.at[wid])

    return deg_kernel


def _make_scatter_kernel(cpt):
    rpt = AGGR // NS

    @functools.partial(
        pl.kernel,
        out_type=jax.ShapeDtypeStruct((NC * AGGR, 128), jnp.float32),
        mesh=_sc_mesh(),
        scratch_types=[
            pltpu.VMEM((cpt, CHUNK), jnp.int32),
            pltpu.VMEM((cpt, CHUNK), jnp.int32),
            pltpu.VMEM((CHUNK, 128), jnp.float32),
            pltpu.VMEM_SHARED((AGGR, 128), jnp.float32),
            pltpu.SemaphoreType.DMA,
        ],
        compiler_params=pltpu.CompilerParams(needs_layout_passes=False),
    )
    def scatter_kernel(y_hbm, src_hbm, dst_hbm, zeros_hbm, out_hbm,
                       srcv, dstv, buf, agg, sem):
        c = lax.axis_index("c")
        s = lax.axis_index("s")
        row0 = (c * NS + s) * cpt
        pltpu.sync_copy(src_hbm.at[pl.ds(row0, cpt)], srcv)
        pltpu.sync_copy(dst_hbm.at[pl.ds(row0, cpt)], dstv)
        pltpu.sync_copy(zeros_hbm.at[pl.ds(0, rpt)], agg.at[pl.ds(s * rpt, rpt)])
        plsc.subcore_barrier()

        def body(i, carry):
            pltpu.async_copy(y_hbm.at[srcv.at[i]], buf, sem).wait()
            pltpu.sync_copy(buf, agg.at[dstv.at[i]], add=True)
            return carry

        lax.fori_loop(0, cpt, body, 0)
        plsc.subcore_barrier()
        pltpu.sync_copy(agg.at[pl.ds(s * rpt, rpt)],
                        out_hbm.at[pl.ds(c * AGGR + s * rpt, rpt)])

    return scatter_kernel


def _dinv_body(cnt_ref, out_ref):
    s = jnp.sum(cnt_ref[...], axis=0, keepdims=True)
    out_ref[...] = lax.rsqrt(1.0 + s)


def _mm_scale_body(h_ref, w_ref, dinv_ref, out_ref):
    out_ref[...] = jnp.dot(h_ref[...], w_ref[...],
                           preferred_element_type=jnp.float32) * dinv_ref[...]


def _bn_relu_mm_body(s_ref, y_ref, dinv_ref, b_ref, g_ref, be_ref, w_ref, out_ref):
    n = y_ref.shape[0]
    agg = dinv_ref[...] * (s_ref[0:n, :] + s_ref[AGGR:AGGR + n, :] + y_ref[...]) \
        + b_ref[...]
    mean = jnp.mean(agg, axis=0, keepdims=True)
    var = jnp.mean((agg - mean) ** 2, axis=0, keepdims=True)
    hn = (agg - mean) * lax.rsqrt(var + 1e-5)
    h = jnp.maximum(g_ref[...] * hn + be_ref[...], 0.0)
    out_ref[...] = jnp.dot(h, w_ref[...],
                           preferred_element_type=jnp.float32) * dinv_ref[...]


def _make_tail_body(num_graphs):
    def tail_body(s_ref, y_ref, dinv_ref, b_ref, g_ref, be_ref, batch_ref,
                  wih_ref, bih_ref, bhh_ref, wfc_ref, bfc_ref, out_ref):
        n = y_ref.shape[0]
        agg = dinv_ref[...] * (s_ref[0:n, :] + s_ref[AGGR:AGGR + n, :] + y_ref[...]) \
            + b_ref[...]
        mean = jnp.mean(agg, axis=0, keepdims=True)
        var = jnp.mean((agg - mean) ** 2, axis=0, keepdims=True)
        hn = (agg - mean) * lax.rsqrt(var + 1e-5)
        h = jnp.maximum(g_ref[...] * hn + be_ref[...], 0.0)

        it = lax.broadcasted_iota(jnp.int32, (num_graphs, n), 0)
        mask = (it == batch_ref[...]).astype(jnp.float32)
        psum = jnp.dot(mask, h, preferred_element_type=jnp.float32)
        cnt = jnp.sum(mask, axis=1, keepdims=True)
        pooled = psum / jnp.maximum(cnt, 1.0)

        lh = wih_ref.shape[1] // 4
        z = jnp.dot(pooled, wih_ref[...], preferred_element_type=jnp.float32) \
            + bih_ref[...] + bhh_ref[...]
        i_g = z[:, 0:lh]
        g_g = z[:, 2 * lh:3 * lh]
        o_g = z[:, 3 * lh:4 * lh]
        cc = jax.nn.sigmoid(i_g) * jnp.tanh(g_g)
        hcell = jax.nn.sigmoid(o_g) * jnp.tanh(cc)
        out_ref[...] = jnp.dot(hcell, wfc_ref[...],
                               preferred_element_type=jnp.float32) + bfc_ref[...]

    return tail_body


def kernel(x, edge_index, batch, W0, b0, g0, be0, W1, b1, g1, be1,
           W2, b2, g2, be2, W_ih, W_hh, b_ih, b_hh, W_fc, b_fc):
    n, d = x.shape
    e = edge_index.shape[1]
    num_graphs = 64

    ept = -(-e // (NW * CHUNK))
    ept = -(-ept // 8) * 8
    ep = ept * NW * CHUNK
    src = edge_index[0]
    dst = edge_index[1]
    pad = ep - e
    src_p = jnp.concatenate([src, jnp.zeros((pad,), jnp.int32)]).reshape(-1, CHUNK)
    dst_p = jnp.concatenate([dst, jnp.full((pad,), n, jnp.int32)]).reshape(-1, CHUNK)
    zeros_hbm = jnp.zeros((AGGR // NS, 128), jnp.float32)

    counts = _make_degree_kernel(ept)(dst_p)
    dinv_row = pl.pallas_call(
        _dinv_body, out_shape=jax.ShapeDtypeStruct((1, AGGR), jnp.float32),
    )(counts)
    dinv = dinv_row.reshape(AGGR, 1)[:n]

    scatter = _make_scatter_kernel(ept)

    def mm_scale(h, w):
        return pl.pallas_call(
            _mm_scale_body, out_shape=jax.ShapeDtypeStruct((n, w.shape[1]), jnp.float32),
        )(h, w, dinv)

    def bn_relu_mm(s_pair, y, b, g, be, w):
        return pl.pallas_call(
            _bn_relu_mm_body,
            out_shape=jax.ShapeDtypeStruct((n, w.shape[1]), jnp.float32),
        )(s_pair, y, dinv, b, g, be, w)

    y1 = mm_scale(x, W0)
    s1 = scatter(y1, src_p, dst_p, zeros_hbm)
    y2 = bn_relu_mm(s1, y1, b0, g0, be0, W1)
    s2 = scatter(y2, src_p, dst_p, zeros_hbm)
    y3 = bn_relu_mm(s2, y2, b1, g1, be1, W2)
    s3 = scatter(y3, src_p, dst_p, zeros_hbm)

    batch2d = batch.reshape(1, n)
    out = pl.pallas_call(
        _make_tail_body(num_graphs),
        out_shape=jax.ShapeDtypeStruct((num_graphs, W_fc.shape[1]), jnp.float32),
    )(s3, y3, dinv, b2, g2, be2, batch2d, W_ih, b_ih, b_hh, W_fc, b_fc)
    return out


# speedup vs baseline: 7.9434x; 1.0817x over previous
import functools

import jax
import jax.numpy as jnp
from jax import lax
from jax.experimental import pallas as pl
from jax.experimental.pallas import tpu as pltpu
from jax.experimental.pallas import tpu_sc as plsc

NC, NS = 2, 16
NW = NC * NS
CHUNK = 128
AGGR = 10112


def _sc_mesh():
    return plsc.VectorSubcoreMesh(core_axis_name="c", subcore_axis_name="s")


def _make_degree_kernel(cpt):
    @functools.partial(
        pl.kernel,
        out_type=jax.ShapeDtypeStruct((NW, AGGR), jnp.float32),
        mesh=_sc_mesh(),
        scratch_types=[
            pltpu.VMEM((cpt, CHUNK), jnp.int32),
            pltpu.VMEM((AGGR,), jnp.float32),
        ],
        compiler_params=pltpu.CompilerParams(needs_layout_passes=False),
    )
    def deg_kernel(dst_hbm, out_hbm, dstv, cnt):
        c = lax.axis_index("c")
        s = lax.axis_index("s")
        wid = c * NS + s
        pltpu.sync_copy(dst_hbm.at[pl.ds(wid * cpt, cpt)], dstv)

        def zero_body(i, carry):
            cnt[pl.ds(i * 16, 16)] = jnp.zeros((16,), jnp.float32)
            return carry

        lax.fori_loop(0, AGGR // 16, zero_body, 0)

        ones = jnp.full((16,), 1.0, jnp.float32)

        def count_body(r, carry):
            for j in range(CHUNK // 16):
                idx = dstv[r, pl.ds(j * 16, 16)]
                plsc.addupdate_scatter(cnt, [idx], ones)
            return carry

        lax.fori_loop(0, cpt, count_body, 0)
        pltpu.sync_copy(cnt, out_hbm.at[wid])

    return deg_kernel


NBUF = 2
NSEG = 2


def _make_scatter_kernel(cpt):
    rpt = AGGR // NS
    seg = cpt // NSEG
    assert cpt == seg * NSEG and seg % NBUF == 0 and seg % 8 == 0

    @functools.partial(
        pl.kernel,
        out_type=jax.ShapeDtypeStruct((NC * AGGR, 128), jnp.float32),
        mesh=_sc_mesh(),
        scratch_types=(
            [pltpu.VMEM((seg, CHUNK), jnp.int32)] * 2
            + [pltpu.VMEM((CHUNK, 128), jnp.float32)] * NBUF
            + [pltpu.SemaphoreType.DMA] * (2 * NBUF)
            + [pltpu.VMEM_SHARED((AGGR, 128), jnp.float32)]
        ),
        compiler_params=pltpu.CompilerParams(needs_layout_passes=False),
    )
    def scatter_kernel(y_hbm, src_hbm, dst_hbm, zeros_hbm, out_hbm,
                       srcv, dstv, *rest):
        bufs = rest[:NBUF]
        gsem = rest[NBUF:2 * NBUF]
        ssem = rest[2 * NBUF:3 * NBUF]
        agg = rest[3 * NBUF]
        c = lax.axis_index("c")
        s = lax.axis_index("s")
        row0 = (c * NS + s) * cpt
        pltpu.sync_copy(zeros_hbm.at[pl.ds(0, rpt)], agg.at[pl.ds(s * rpt, rpt)])
        plsc.subcore_barrier()

        def gstart(i, b):
            pltpu.async_copy(y_hbm.at[srcv.at[i]], bufs[b], gsem[b])

        def gwait(i, b):
            pltpu.make_async_copy(y_hbm.at[srcv.at[i]], bufs[b], gsem[b]).wait()

        def sstart(i, b):
            pltpu.async_copy(bufs[b], agg.at[dstv.at[i]], ssem[b], add=True)

        def swait(i, b):
            pltpu.make_async_copy(bufs[b], agg.at[dstv.at[i]], ssem[b]).wait()

        for sg in range(NSEG):
            pltpu.sync_copy(src_hbm.at[pl.ds(row0 + sg * seg, seg)], srcv)
            pltpu.sync_copy(dst_hbm.at[pl.ds(row0 + sg * seg, seg)], dstv)
            for b in range(NBUF):
                gstart(b, b)

            def group(gi, carry):
                i0 = gi * NBUF
                for b in range(NBUF):
                    gwait(i0 + b, b)
                    sstart(i0 + b, b)
                for b in range(NBUF):
                    nxt = i0 + b + NBUF

                    @pl.when(nxt < seg)
                    def _():
                        swait(i0 + b, b)
                        gstart(nxt, b)

                return carry

            lax.fori_loop(0, seg // NBUF, group, 0)
            for b in range(NBUF):
                swait(seg - NBUF + b, b)

        plsc.subcore_barrier()
        pltpu.sync_copy(agg.at[pl.ds(s * rpt, rpt)],
                        out_hbm.at[pl.ds(c * AGGR + s * rpt, rpt)])

    return scatter_kernel


def _dinv_body(cnt_ref, out_ref):
    s = jnp.sum(cnt_ref[...], axis=0, keepdims=True)
    out_ref[...] = lax.rsqrt(1.0 + s)


def _mm_scale_body(h_ref, w_ref, dinv_ref, out_ref):
    out_ref[...] = jnp.dot(h_ref[...], w_ref[...],
                           preferred_element_type=jnp.float32) * dinv_ref[...]


def _bn_relu_mm_body(s_ref, y_ref, dinv_ref, b_ref, g_ref, be_ref, w_ref, out_ref):
    n = y_ref.shape[0]
    agg = dinv_ref[...] * (s_ref[0:n, :] + s_ref[AGGR:AGGR + n, :] + y_ref[...]) \
        + b_ref[...]
    mean = jnp.mean(agg, axis=0, keepdims=True)
    var = jnp.mean((agg - mean) ** 2, axis=0, keepdims=True)
    hn = (agg - mean) * lax.rsqrt(var + 1e-5)
    h = jnp.maximum(g_ref[...] * hn + be_ref[...], 0.0)
    out_ref[...] = jnp.dot(h, w_ref[...],
                           preferred_element_type=jnp.float32) * dinv_ref[...]


def _make_tail_body(num_graphs):
    def tail_body(s_ref, y_ref, dinv_ref, b_ref, g_ref, be_ref, batch_ref,
                  wih_ref, bih_ref, bhh_ref, wfc_ref, bfc_ref, out_ref):
        n = y_ref.shape[0]
        agg = dinv_ref[...] * (s_ref[0:n, :] + s_ref[AGGR:AGGR + n, :] + y_ref[...]) \
            + b_ref[...]
        mean = jnp.mean(agg, axis=0, keepdims=True)
        var = jnp.mean((agg - mean) ** 2, axis=0, keepdims=True)
        hn = (agg - mean) * lax.rsqrt(var + 1e-5)
        h = jnp.maximum(g_ref[...] * hn + be_ref[...], 0.0)

        it = lax.broadcasted_iota(jnp.int32, (num_graphs, n), 0)
        mask = (it == batch_ref[...]).astype(jnp.float32)
        psum = jnp.dot(mask, h, preferred_element_type=jnp.float32)
        cnt = jnp.sum(mask, axis=1, keepdims=True)
        pooled = psum / jnp.maximum(cnt, 1.0)

        lh = wih_ref.shape[1] // 4
        z = jnp.dot(pooled, wih_ref[...], preferred_element_type=jnp.float32) \
            + bih_ref[...] + bhh_ref[...]
        i_g = z[:, 0:lh]
        g_g = z[:, 2 * lh:3 * lh]
        o_g = z[:, 3 * lh:4 * lh]
        cc = jax.nn.sigmoid(i_g) * jnp.tanh(g_g)
        hcell = jax.nn.sigmoid(o_g) * jnp.tanh(cc)
        out_ref[...] = jnp.dot(hcell, wfc_ref[...],
                               preferred_element_type=jnp.float32) + bfc_ref[...]

    return tail_body


def kernel(x, edge_index, batch, W0, b0, g0, be0, W1, b1, g1, be1,
           W2, b2, g2, be2, W_ih, W_hh, b_ih, b_hh, W_fc, b_fc):
    n, d = x.shape
    e = edge_index.shape[1]
    num_graphs = 64

    ept = -(-e // (NW * CHUNK))
    ept = -(-ept // 8) * 8
    ep = ept * NW * CHUNK
    src = edge_index[0]
    dst = edge_index[1]
    pad = ep - e
    src_p = jnp.concatenate([src, jnp.zeros((pad,), jnp.int32)]).reshape(-1, CHUNK)
    dst_p = jnp.concatenate([dst, jnp.full((pad,), n, jnp.int32)]).reshape(-1, CHUNK)
    zeros_hbm = jnp.zeros((AGGR // NS, 128), jnp.float32)

    counts = _make_degree_kernel(ept)(dst_p)
    dinv_row = pl.pallas_call(
        _dinv_body, out_shape=jax.ShapeDtypeStruct((1, AGGR), jnp.float32),
    )(counts)
    dinv = dinv_row.reshape(AGGR, 1)[:n]

    scatter = _make_scatter_kernel(ept)

    def mm_scale(h, w):
        return pl.pallas_call(
            _mm_scale_body, out_shape=jax.ShapeDtypeStruct((n, w.shape[1]), jnp.float32),
        )(h, w, dinv)

    def bn_relu_mm(s_pair, y, b, g, be, w):
        return pl.pallas_call(
            _bn_relu_mm_body,
            out_shape=jax.ShapeDtypeStruct((n, w.shape[1]), jnp.float32),
        )(s_pair, y, dinv, b, g, be, w)

    y1 = mm_scale(x, W0)
    s1 = scatter(y1, src_p, dst_p, zeros_hbm)
    y2 = bn_relu_mm(s1, y1, b0, g0, be0, W1)
    s2 = scatter(y2, src_p, dst_p, zeros_hbm)
    y3 = bn_relu_mm(s2, y2, b1, g1, be1, W2)
    s3 = scatter(y3, src_p, dst_p, zeros_hbm)

    batch2d = batch.reshape(1, n)
    out = pl.pallas_call(
        _make_tail_body(num_graphs),
        out_shape=jax.ShapeDtypeStruct((num_graphs, W_fc.shape[1]), jnp.float32),
    )(s3, y3, dinv, b2, g2, be2, batch2d, W_ih, b_ih, b_hh, W_fc, b_fc)
    return out


# speedup vs baseline: 22.7840x; 2.8683x over previous
import functools

import jax
import jax.numpy as jnp
from jax import lax
from jax.experimental import pallas as pl
from jax.experimental.pallas import tpu as pltpu
from jax.experimental.pallas import tpu_sc as plsc

NC, NS = 2, 16
NW = NC * NS
CHUNK = 128
AGGR = 10112


def _sc_mesh():
    return plsc.VectorSubcoreMesh(core_axis_name="c", subcore_axis_name="s")


def _make_degree_kernel(cpt):
    @functools.partial(
        pl.kernel,
        out_type=jax.ShapeDtypeStruct((NW, AGGR), jnp.float32),
        mesh=_sc_mesh(),
        scratch_types=[
            pltpu.VMEM((cpt, CHUNK), jnp.int32),
            pltpu.VMEM((AGGR,), jnp.float32),
        ],
        compiler_params=pltpu.CompilerParams(needs_layout_passes=False),
    )
    def deg_kernel(dst_hbm, out_hbm, dstv, cnt):
        c = lax.axis_index("c")
        s = lax.axis_index("s")
        wid = c * NS + s
        pltpu.sync_copy(dst_hbm.at[pl.ds(wid * cpt, cpt)], dstv)

        def zero_body(i, carry):
            cnt[pl.ds(i * 16, 16)] = jnp.zeros((16,), jnp.float32)
            return carry

        lax.fori_loop(0, AGGR // 16, zero_body, 0)

        ones = jnp.full((16,), 1.0, jnp.float32)

        def count_body(r, carry):
            for j in range(CHUNK // 16):
                idx = dstv[r, pl.ds(j * 16, 16)]
                plsc.addupdate_scatter(cnt, [idx], ones)
            return carry

        lax.fori_loop(0, cpt, count_body, 0)
        pltpu.sync_copy(cnt, out_hbm.at[wid])

    return deg_kernel


NBUF = 2
NSEG = 2


def _make_scatter_kernel(cpt):
    rpt = AGGR // NS
    seg = cpt // NSEG
    assert cpt == seg * NSEG and seg % NBUF == 0 and seg % 8 == 0

    @functools.partial(
        pl.kernel,
        out_type=jax.ShapeDtypeStruct((NC * AGGR, 128), jnp.float32),
        mesh=_sc_mesh(),
        scratch_types=(
            [pltpu.VMEM((seg, CHUNK), jnp.int32)] * 2
            + [pltpu.VMEM((CHUNK, 128), jnp.float32)] * NBUF
            + [pltpu.SemaphoreType.DMA] * (2 * NBUF)
            + [pltpu.VMEM_SHARED((AGGR, 128), jnp.float32)]
        ),
        compiler_params=pltpu.CompilerParams(needs_layout_passes=False),
    )
    def scatter_kernel(y_hbm, src_hbm, dst_hbm, zeros_hbm, out_hbm,
                       srcv, dstv, *rest):
        bufs = rest[:NBUF]
        gsem = rest[NBUF:2 * NBUF]
        ssem = rest[2 * NBUF:3 * NBUF]
        agg = rest[3 * NBUF]
        c = lax.axis_index("c")
        s = lax.axis_index("s")
        row0 = (c * NS + s) * cpt
        pltpu.sync_copy(zeros_hbm.at[pl.ds(0, rpt)], agg.at[pl.ds(s * rpt, rpt)])
        plsc.subcore_barrier()

        def gstart(i, b):
            pltpu.async_copy(y_hbm.at[srcv.at[i]], bufs[b], gsem[b])

        def gwait(i, b):
            pltpu.make_async_copy(y_hbm.at[srcv.at[i]], bufs[b], gsem[b]).wait()

        def sstart(i, b):
            pltpu.async_copy(bufs[b], agg.at[dstv.at[i]], ssem[b], add=True)

        def swait(i, b):
            pltpu.make_async_copy(bufs[b], agg.at[dstv.at[i]], ssem[b]).wait()

        for sg in range(NSEG):
            pltpu.sync_copy(src_hbm.at[pl.ds(row0 + sg * seg, seg)], srcv)
            pltpu.sync_copy(dst_hbm.at[pl.ds(row0 + sg * seg, seg)], dstv)
            for b in range(NBUF):
                gstart(b, b)

            def group(gi, carry):
                i0 = gi * NBUF
                for b in range(NBUF):
                    gwait(i0 + b, b)
                    sstart(i0 + b, b)
                for b in range(NBUF):
                    nxt = i0 + b + NBUF

                    @pl.when(nxt < seg)
                    def _():
                        swait(i0 + b, b)
                        gstart(nxt, b)

                return carry

            lax.fori_loop(0, seg // NBUF, group, 0)
            for b in range(NBUF):
                swait(seg - NBUF + b, b)

        plsc.subcore_barrier()
        pltpu.sync_copy(agg.at[pl.ds(s * rpt, rpt)],
                        out_hbm.at[pl.ds(c * AGGR + s * rpt, rpt)])

    return scatter_kernel


def _dinv_body(cnt_ref, out_ref):
    s = jnp.sum(cnt_ref[...], axis=0, keepdims=True)
    out_ref[...] = lax.rsqrt(1.0 + s)


def _mm_scale_body(h_ref, w_ref, dinv_ref, out_ref):
    out_ref[...] = jnp.dot(h_ref[...], w_ref[...],
                           preferred_element_type=jnp.float32) * dinv_ref[...]


def _bn_relu_mm_body(s_ref, y_ref, dinv_ref, b_ref, g_ref, be_ref, w_ref, out_ref):
    n = y_ref.shape[0]
    agg = dinv_ref[...] * (s_ref[0:n, :] + s_ref[AGGR:AGGR + n, :] + y_ref[...]) \
        + b_ref[...]
    mean = jnp.mean(agg, axis=0, keepdims=True)
    var = jnp.mean((agg - mean) ** 2, axis=0, keepdims=True)
    hn = (agg - mean) * lax.rsqrt(var + 1e-5)
    h = jnp.maximum(g_ref[...] * hn + be_ref[...], 0.0)
    out_ref[...] = jnp.dot(h, w_ref[...],
                           preferred_element_type=jnp.float32) * dinv_ref[...]


def _make_tail_body(num_graphs):
    def tail_body(s_ref, y_ref, dinv_ref, b_ref, g_ref, be_ref, batch_ref,
                  wih_ref, bih_ref, bhh_ref, wfc_ref, bfc_ref, out_ref):
        n = y_ref.shape[0]
        agg = dinv_ref[...] * (s_ref[0:n, :] + s_ref[AGGR:AGGR + n, :] + y_ref[...]) \
            + b_ref[...]
        mean = jnp.mean(agg, axis=0, keepdims=True)
        var = jnp.mean((agg - mean) ** 2, axis=0, keepdims=True)
        hn = (agg - mean) * lax.rsqrt(var + 1e-5)
        h = jnp.maximum(g_ref[...] * hn + be_ref[...], 0.0)

        it = lax.broadcasted_iota(jnp.int32, (num_graphs, n), 0)
        mask = (it == batch_ref[...]).astype(jnp.float32)
        psum = jnp.dot(mask, h, preferred_element_type=jnp.float32)
        cnt = jnp.sum(mask, axis=1, keepdims=True)
        pooled = psum / jnp.maximum(cnt, 1.0)

        lh = wih_ref.shape[1] // 4
        z = jnp.dot(pooled, wih_ref[...], preferred_element_type=jnp.float32) \
            + bih_ref[...] + bhh_ref[...]
        i_g = z[:, 0:lh]
        g_g = z[:, 2 * lh:3 * lh]
        o_g = z[:, 3 * lh:4 * lh]
        cc = jax.nn.sigmoid(i_g) * jnp.tanh(g_g)
        hcell = jax.nn.sigmoid(o_g) * jnp.tanh(cc)
        out_ref[...] = jnp.dot(hcell, wfc_ref[...],
                               preferred_element_type=jnp.float32) + bfc_ref[...]

    return tail_body


def kernel(x, edge_index, batch, W0, b0, g0, be0, W1, b1, g1, be1,
           W2, b2, g2, be2, W_ih, W_hh, b_ih, b_hh, W_fc, b_fc):
    n, d = x.shape
    e = edge_index.shape[1]
    num_graphs = 64

    ept = -(-e // (NW * CHUNK))
    ept = -(-ept // 8) * 8
    ep = ept * NW * CHUNK
    src = edge_index[0]
    dst = edge_index[1]
    pad = ep - e
    pad_i = jnp.arange(pad, dtype=jnp.int32)
    src_p = jnp.concatenate([src, pad_i % n]).reshape(-1, CHUNK)
    dst_p = jnp.concatenate([dst, n + pad_i % (AGGR - n)]).reshape(-1, CHUNK)
    zeros_hbm = jnp.zeros((AGGR // NS, 128), jnp.float32)

    counts = _make_degree_kernel(ept)(dst_p)
    dinv_row = pl.pallas_call(
        _dinv_body, out_shape=jax.ShapeDtypeStruct((1, AGGR), jnp.float32),
    )(counts)
    dinv = dinv_row.reshape(AGGR, 1)[:n]

    scatter = _make_scatter_kernel(ept)

    def mm_scale(h, w):
        return pl.pallas_call(
            _mm_scale_body, out_shape=jax.ShapeDtypeStruct((n, w.shape[1]), jnp.float32),
        )(h, w, dinv)

    def bn_relu_mm(s_pair, y, b, g, be, w):
        return pl.pallas_call(
            _bn_relu_mm_body,
            out_shape=jax.ShapeDtypeStruct((n, w.shape[1]), jnp.float32),
        )(s_pair, y, dinv, b, g, be, w)

    y1 = mm_scale(x, W0)
    s1 = scatter(y1, src_p, dst_p, zeros_hbm)
    y2 = bn_relu_mm(s1, y1, b0, g0, be0, W1)
    s2 = scatter(y2, src_p, dst_p, zeros_hbm)
    y3 = bn_relu_mm(s2, y2, b1, g1, be1, W2)
    s3 = scatter(y3, src_p, dst_p, zeros_hbm)

    batch2d = batch.reshape(1, n)
    out = pl.pallas_call(
        _make_tail_body(num_graphs),
        out_shape=jax.ShapeDtypeStruct((num_graphs, W_fc.shape[1]), jnp.float32),
    )(s3, y3, dinv, b2, g2, be2, batch2d, W_ih, b_ih, b_hh, W_fc, b_fc)
    return out


# speedup vs baseline: 24.0638x; 1.0562x over previous
import functools

import jax
import jax.numpy as jnp
from jax import lax
from jax.experimental import pallas as pl
from jax.experimental.pallas import tpu as pltpu
from jax.experimental.pallas import tpu_sc as plsc

NC, NS = 2, 16
NW = NC * NS
CHUNK = 112
AGGR = 10112


def _sc_mesh():
    return plsc.VectorSubcoreMesh(core_axis_name="c", subcore_axis_name="s")


def _make_degree_kernel(cpt):
    @functools.partial(
        pl.kernel,
        out_type=jax.ShapeDtypeStruct((NW, AGGR), jnp.float32),
        mesh=_sc_mesh(),
        scratch_types=[
            pltpu.VMEM((cpt, CHUNK), jnp.int32),
            pltpu.VMEM((AGGR,), jnp.float32),
        ],
        compiler_params=pltpu.CompilerParams(needs_layout_passes=False),
    )
    def deg_kernel(dst_hbm, out_hbm, dstv, cnt):
        c = lax.axis_index("c")
        s = lax.axis_index("s")
        wid = c * NS + s
        pltpu.sync_copy(dst_hbm.at[pl.ds(wid * cpt, cpt)], dstv)

        def zero_body(i, carry):
            cnt[pl.ds(i * 16, 16)] = jnp.zeros((16,), jnp.float32)
            return carry

        lax.fori_loop(0, AGGR // 16, zero_body, 0)

        ones = jnp.full((16,), 1.0, jnp.float32)

        def count_body(r, carry):
            for j in range(CHUNK // 16):
                idx = dstv[r, pl.ds(j * 16, 16)]
                plsc.addupdate_scatter(cnt, [idx], ones)
            return carry

        lax.fori_loop(0, cpt, count_body, 0)
        pltpu.sync_copy(cnt, out_hbm.at[wid])

    return deg_kernel


NBUF = 3
NSEG = 4


def _make_scatter_kernel(cpt):
    rpt = AGGR // NS
    seg = cpt // NSEG
    assert cpt == seg * NSEG and seg % NBUF == 0 and seg % 8 == 0

    @functools.partial(
        pl.kernel,
        out_type=jax.ShapeDtypeStruct((NC * AGGR, 128), jnp.float32),
        mesh=_sc_mesh(),
        scratch_types=(
            [pltpu.VMEM((seg, CHUNK), jnp.int32)] * 2
            + [pltpu.VMEM((CHUNK, 128), jnp.float32)] * NBUF
            + [pltpu.SemaphoreType.DMA] * (2 * NBUF)
            + [pltpu.VMEM_SHARED((AGGR, 128), jnp.float32)]
        ),
        compiler_params=pltpu.CompilerParams(needs_layout_passes=False),
    )
    def scatter_kernel(y_hbm, src_hbm, dst_hbm, zeros_hbm, out_hbm,
                       srcv, dstv, *rest):
        bufs = rest[:NBUF]
        gsem = rest[NBUF:2 * NBUF]
        ssem = rest[2 * NBUF:3 * NBUF]
        agg = rest[3 * NBUF]
        c = lax.axis_index("c")
        s = lax.axis_index("s")
        row0 = (c * NS + s) * cpt
        pltpu.sync_copy(zeros_hbm.at[pl.ds(0, rpt)], agg.at[pl.ds(s * rpt, rpt)])
        plsc.subcore_barrier()

        def gstart(i, b):
            pltpu.async_copy(y_hbm.at[srcv.at[i]], bufs[b], gsem[b])

        def gwait(i, b):
            pltpu.make_async_copy(y_hbm.at[srcv.at[i]], bufs[b], gsem[b]).wait()

        def sstart(i, b):
            pltpu.async_copy(bufs[b], agg.at[dstv.at[i]], ssem[b], add=True)

        def swait(i, b):
            pltpu.make_async_copy(bufs[b], agg.at[dstv.at[i]], ssem[b]).wait()

        for sg in range(NSEG):
            pltpu.sync_copy(src_hbm.at[pl.ds(row0 + sg * seg, seg)], srcv)
            pltpu.sync_copy(dst_hbm.at[pl.ds(row0 + sg * seg, seg)], dstv)
            for b in range(NBUF):
                gstart(b, b)

            def group(gi, carry):
                i0 = gi * NBUF
                for b in range(NBUF):
                    gwait(i0 + b, b)
                    sstart(i0 + b, b)
                for b in range(NBUF):
                    nxt = i0 + b + NBUF

                    @pl.when(nxt < seg)
                    def _():
                        swait(i0 + b, b)
                        gstart(nxt, b)

                return carry

            lax.fori_loop(0, seg // NBUF, group, 0)
            for b in range(NBUF):
                swait(seg - NBUF + b, b)

        plsc.subcore_barrier()
        pltpu.sync_copy(agg.at[pl.ds(s * rpt, rpt)],
                        out_hbm.at[pl.ds(c * AGGR + s * rpt, rpt)])

    return scatter_kernel


def _dinv_body(cnt_ref, out_ref):
    s = jnp.sum(cnt_ref[...], axis=0, keepdims=True)
    out_ref[...] = lax.rsqrt(1.0 + s)


def _mm_scale_body(h_ref, w_ref, dinv_ref, out_ref):
    out_ref[...] = jnp.dot(h_ref[...], w_ref[...],
                           preferred_element_type=jnp.float32) * dinv_ref[...]


def _bn_relu_mm_body(s_ref, y_ref, dinv_ref, b_ref, g_ref, be_ref, w_ref, out_ref):
    n = y_ref.shape[0]
    agg = dinv_ref[...] * (s_ref[0:n, :] + s_ref[AGGR:AGGR + n, :] + y_ref[...]) \
        + b_ref[...]
    mean = jnp.mean(agg, axis=0, keepdims=True)
    var = jnp.mean((agg - mean) ** 2, axis=0, keepdims=True)
    hn = (agg - mean) * lax.rsqrt(var + 1e-5)
    h = jnp.maximum(g_ref[...] * hn + be_ref[...], 0.0)
    out_ref[...] = jnp.dot(h, w_ref[...],
                           preferred_element_type=jnp.float32) * dinv_ref[...]


def _make_tail_body(num_graphs):
    def tail_body(s_ref, y_ref, dinv_ref, b_ref, g_ref, be_ref, batch_ref,
                  wih_ref, bih_ref, bhh_ref, wfc_ref, bfc_ref, out_ref):
        n = y_ref.shape[0]
        agg = dinv_ref[...] * (s_ref[0:n, :] + s_ref[AGGR:AGGR + n, :] + y_ref[...]) \
            + b_ref[...]
        mean = jnp.mean(agg, axis=0, keepdims=True)
        var = jnp.mean((agg - mean) ** 2, axis=0, keepdims=True)
        hn = (agg - mean) * lax.rsqrt(var + 1e-5)
        h = jnp.maximum(g_ref[...] * hn + be_ref[...], 0.0)

        it = lax.broadcasted_iota(jnp.int32, (num_graphs, n), 0)
        mask = (it == batch_ref[...]).astype(jnp.float32)
        psum = jnp.dot(mask, h, preferred_element_type=jnp.float32)
        cnt = jnp.sum(mask, axis=1, keepdims=True)
        pooled = psum / jnp.maximum(cnt, 1.0)

        lh = wih_ref.shape[1] // 4
        z = jnp.dot(pooled, wih_ref[...], preferred_element_type=jnp.float32) \
            + bih_ref[...] + bhh_ref[...]
        i_g = z[:, 0:lh]
        g_g = z[:, 2 * lh:3 * lh]
        o_g = z[:, 3 * lh:4 * lh]
        cc = jax.nn.sigmoid(i_g) * jnp.tanh(g_g)
        hcell = jax.nn.sigmoid(o_g) * jnp.tanh(cc)
        out_ref[...] = jnp.dot(hcell, wfc_ref[...],
                               preferred_element_type=jnp.float32) + bfc_ref[...]

    return tail_body


def kernel(x, edge_index, batch, W0, b0, g0, be0, W1, b1, g1, be1,
           W2, b2, g2, be2, W_ih, W_hh, b_ih, b_hh, W_fc, b_fc):
    n, d = x.shape
    e = edge_index.shape[1]
    num_graphs = 64

    ept = -(-e // (NW * CHUNK))
    ept = -(-ept // 24) * 24
    ep = ept * NW * CHUNK
    src = edge_index[0]
    dst = edge_index[1]
    pad = ep - e
    pad_i = jnp.arange(pad, dtype=jnp.int32)
    src_p = jnp.concatenate([src, pad_i % n]).reshape(-1, CHUNK)
    dst_p = jnp.concatenate([dst, n + pad_i % (AGGR - n)]).reshape(-1, CHUNK)
    zeros_hbm = jnp.zeros((AGGR // NS, 128), jnp.float32)

    counts = _make_degree_kernel(ept)(dst_p)
    dinv_row = pl.pallas_call(
        _dinv_body, out_shape=jax.ShapeDtypeStruct((1, AGGR), jnp.float32),
    )(counts)
    dinv = dinv_row.reshape(AGGR, 1)[:n]

    scatter = _make_scatter_kernel(ept)

    def mm_scale(h, w):
        return pl.pallas_call(
            _mm_scale_body, out_shape=jax.ShapeDtypeStruct((n, w.shape[1]), jnp.float32),
        )(h, w, dinv)

    def bn_relu_mm(s_pair, y, b, g, be, w):
        return pl.pallas_call(
            _bn_relu_mm_body,
            out_shape=jax.ShapeDtypeStruct((n, w.shape[1]), jnp.float32),
        )(s_pair, y, dinv, b, g, be, w)

    y1 = mm_scale(x, W0)
    s1 = scatter(y1, src_p, dst_p, zeros_hbm)
    y2 = bn_relu_mm(s1, y1, b0, g0, be0, W1)
    s2 = scatter(y2, src_p, dst_p, zeros_hbm)
    y3 = bn_relu_mm(s2, y2, b1, g1, be1, W2)
    s3 = scatter(y3, src_p, dst_p, zeros_hbm)

    batch2d = batch.reshape(1, n)
    out = pl.pallas_call(
        _make_tail_body(num_graphs),
        out_shape=jax.ShapeDtypeStruct((num_graphs, W_fc.shape[1]), jnp.float32),
    )(s3, y3, dinv, b2, g2, be2, batch2d, W_ih, b_ih, b_hh, W_fc, b_fc)
    return out
